# baseline (speedup 1.0000x reference)
import jax
import jax.numpy as jnp
from jax import lax
from jax.experimental import pallas as pl
from jax.experimental.pallas import tpu as pltpu
from jax.experimental.pallas import tpu_sc as plsc

F32 = jnp.float32
I32 = jnp.int32
HIGH = lax.Precision.HIGHEST

NC, NS = 2, 16
NW = NC * NS
IDXW = 80
GB = 256
ZROWS = 64


def _sc_mesh():
    return plsc.VectorSubcoreMesh(
        core_axis_name="c", subcore_axis_name="s",
        num_cores=NC, num_subcores=NS)


def _sc_gather(table, idx3d):
    _, chunks, w = idx3d.shape
    e_total = NW * chunks * w
    d = table.shape[1]
    rows_w = chunks * w

    u = 8
    main = chunks // u
    tail = chunks % u

    def body(table_ref, idx_ref, out_ref, idx_v, rows_v, *sems):
        gsems = sems[:u]
        wsem = sems[u]
        wid = lax.axis_index("c") * NS + lax.axis_index("s")
        pltpu.sync_copy(idx_ref.at[wid], idx_v)
        ebase = wid * rows_w

        def group(j2, carry):
            j0 = j2 * u
            gds = [pltpu.async_copy(table_ref.at[idx_v.at[j0 + b]],
                                    rows_v.at[b], gsems[b])
                   for b in range(u)]
            wds = []
            for b in range(u):
                gds[b].wait()
                wds.append(pltpu.async_copy(
                    rows_v.at[b], out_ref.at[pl.ds(ebase + (j0 + b) * w, w)],
                    wsem))
            for d_ in wds:
                d_.wait()
            return carry

        lax.fori_loop(0, main, group, 0)
        for t in range(tail):
            j = main * u + t
            pltpu.async_copy(table_ref.at[idx_v.at[j]], rows_v.at[0],
                             gsems[0]).wait()
            pltpu.sync_copy(rows_v.at[0], out_ref.at[pl.ds(ebase + j * w, w)])

    return pl.kernel(
        body,
        out_type=jax.ShapeDtypeStruct((e_total, d), F32),
        mesh=_sc_mesh(),
        scratch_types=[
            pltpu.VMEM((chunks, w), I32),
            pltpu.VMEM((u, w, d), F32),
        ] + [pltpu.SemaphoreType.DMA] * (u + 1),
    )(table, idx3d)


def _sc_scatter(e, egi3d, rcv3d, zd, n_acc, with_recv):
    _, chunks, w = egi3d.shape
    d = e.shape[1]
    rows_w = chunks * w
    npt = n_acc // NS
    nz = npt // w
    gpt = GB // NS

    u = 3 if with_recv else 5
    main = chunks // u
    tail = chunks % u

    out_type = [jax.ShapeDtypeStruct((NC, GB, d), F32)]
    scratch = [
        pltpu.VMEM((u, w, d), F32),
        pltpu.VMEM((u, 1, w), I32),
        pltpu.VMEM_SHARED((GB, d), F32),
    ]
    if with_recv:
        out_type.append(jax.ShapeDtypeStruct((NC, n_acc, d), F32))
        scratch += [pltpu.VMEM((u, 1, w), I32),
                    pltpu.VMEM_SHARED((n_acc, d), F32)]
    scratch += [pltpu.SemaphoreType.DMA] * (2 * u)

    def body(e_ref, egi_ref, rcv_ref, zd_ref, *rest):
        rest = list(rest)
        q_ref = rest.pop(0)
        p_ref = rest.pop(0) if with_recv else None
        rows = rest.pop(0)
        idx_g = rest.pop(0)
        acc_g = rest.pop(0)
        if with_recv:
            idx_r = rest.pop(0)
            acc_n = rest.pop(0)
        lsems = rest[:u]
        asems = rest[u:2 * u]

        cid = lax.axis_index("c")
        sid = lax.axis_index("s")
        wid = cid * NS + sid

        pltpu.sync_copy(zd_ref, rows.at[0])
        pltpu.sync_copy(rows.at[0, pl.ds(0, gpt)],
                        acc_g.at[pl.ds(sid * gpt, gpt)])
        if with_recv:
            for z in range(nz):
                pltpu.sync_copy(rows.at[0], acc_n.at[pl.ds(sid * npt + z * w, w)])
        plsc.subcore_barrier()

        ebase = wid * rows_w

        def issue_loads(j, b):
            lds = [pltpu.async_copy(e_ref.at[pl.ds(ebase + j * w, w)],
                                    rows.at[b], lsems[b]),
                   pltpu.async_copy(egi_ref.at[wid, pl.ds(j, 1)],
                                    idx_g.at[b], lsems[b])]
            if with_recv:
                lds.append(pltpu.async_copy(rcv_ref.at[wid, pl.ds(j, 1)],
                                            idx_r.at[b], lsems[b]))
            return lds

        def issue_adds(b):
            ads = [pltpu.async_copy(rows.at[b], acc_g.at[idx_g.at[b, 0]],
                                    asems[b], add=True)]
            if with_recv:
                ads.append(pltpu.async_copy(rows.at[b],
                                            acc_n.at[idx_r.at[b, 0]],
                                            asems[b], add=True))
            return ads

        def group(j2, carry):
            j0 = j2 * u
            lds = [issue_loads(j0 + b, b) for b in range(u)]
            ads = []
            for b in range(u):
                for l_ in lds[b]:
                    l_.wait()
                ads += issue_adds(b)
            for a_ in ads:
                a_.wait()
            return carry

        lax.fori_loop(0, main, group, 0)
        for t in range(tail):
            for l_ in issue_loads(main * u + t, 0):
                l_.wait()
            for a_ in issue_adds(0):
                a_.wait()
        plsc.subcore_barrier()

        pltpu.sync_copy(acc_g.at[pl.ds(sid * gpt, gpt)],
                        rows.at[0, pl.ds(0, gpt)])
        pltpu.sync_copy(rows.at[0, pl.ds(0, gpt)],
                        q_ref.at[cid, pl.ds(sid * gpt, gpt)])
        if with_recv:
            for z in range(nz):
                r0 = sid * npt + z * w
                pltpu.sync_copy(acc_n.at[pl.ds(r0, w)], rows.at[0])
                pltpu.sync_copy(rows.at[0], p_ref.at[cid, pl.ds(r0, w)])

    res = pl.kernel(
        body,
        out_type=out_type,
        mesh=_sc_mesh(),
        scratch_types=scratch,
    )(e, egi3d, rcv3d, zd)
    return res


def _sc_counts(rcv3d, zd, onesb, n_acc):
    _, chunks, w = rcv3d.shape
    d = onesb.shape[1]
    npt = n_acc // NS
    nz = npt // w

    def body(rcv_ref, zd_ref, ones_ref, rc_ref, idx_r, buf, acc_c):
        cid = lax.axis_index("c")
        sid = lax.axis_index("s")
        wid = cid * NS + sid
        pltpu.sync_copy(zd_ref, buf)
        for z in range(nz):
            pltpu.sync_copy(buf, acc_c.at[pl.ds(sid * npt + z * w, w)])
        pltpu.sync_copy(ones_ref, buf)
        pltpu.sync_copy(rcv_ref.at[wid], idx_r)
        plsc.subcore_barrier()

        def step(j, carry):
            pltpu.sync_copy(buf, acc_c.at[idx_r.at[j]], add=True)
            return carry

        lax.fori_loop(0, chunks, step, 0)
        plsc.subcore_barrier()
        for z in range(nz):
            r0 = sid * npt + z * w
            pltpu.sync_copy(acc_c.at[pl.ds(r0, w)], buf)
            pltpu.sync_copy(buf, rc_ref.at[cid, pl.ds(r0, w)])

    return pl.kernel(
        body,
        out_type=jax.ShapeDtypeStruct((NC, n_acc, d), F32),
        mesh=_sc_mesh(),
        scratch_types=[
            pltpu.VMEM((chunks, w), I32),
            pltpu.VMEM((w, d), F32),
            pltpu.VMEM_SHARED((n_acc, d), F32),
        ],
    )(rcv3d, zd, onesb)


def _dot(a, b):
    return jnp.dot(a, b, preferred_element_type=F32)


def _onehot_t(ids, nrows):
    return (jnp.broadcast_to(ids, (nrows, GB))
            == lax.broadcasted_iota(I32, (nrows, GB), 1)).astype(F32)


def _seg_dot(ohT, x):
    return lax.dot_general(ohT, x, (((0,), (0,)), ((), ())),
                           preferred_element_type=F32, precision=HIGH)


def _tc_enc_edge(ef, egi_col, We, be_row, be_blk):
    e_total, de = ef.shape
    he = We.shape[1]
    grid = e_total // be_blk

    def body(ef_ref, egi_ref, We_ref, be_ref, e0_ref, ecnt_ref):
        x = _dot(ef_ref[...], We_ref[...])
        e0_ref[...] = jnp.maximum(x + be_ref[...], 0.0)
        ohT = _onehot_t(egi_ref[...], be_blk)
        cnt = _seg_dot(ohT, jnp.ones((be_blk, 8), F32))

        @pl.when(pl.program_id(0) == 0)
        def _():
            ecnt_ref[...] = jnp.zeros_like(ecnt_ref)

        ecnt_ref[...] += cnt

    return pl.pallas_call(
        body,
        grid=(grid,),
        in_specs=[
            pl.BlockSpec((be_blk, de), lambda j: (j, 0)),
            pl.BlockSpec((be_blk, 1), lambda j: (j, 0)),
            pl.BlockSpec((de, he), lambda j: (0, 0)),
            pl.BlockSpec((1, he), lambda j: (0, 0)),
        ],
        out_specs=[
            pl.BlockSpec((be_blk, he), lambda j: (j, 0)),
            pl.BlockSpec((GB, 8), lambda j: (0, 0)),
        ],
        out_shape=[
            jax.ShapeDtypeStruct((e_total, he), F32),
            jax.ShapeDtypeStruct((GB, 8), F32),
        ],
    )(ef, egi_col, We, be_row)


def _tc_enc_node(nf, ngi_col, Wn, bn_row, Wes0, bn_blk):
    n_total, dn = nf.shape
    h = Wn.shape[1]
    he = Wes0.shape[1]
    grid = n_total // bn_blk

    def body(nf_ref, ngi_ref, Wn_ref, bn_ref, Wes_ref,
             n0_ref, m0_ref, sn_ref, ncnt_ref):
        n = jnp.maximum(_dot(nf_ref[...], Wn_ref[...]) + bn_ref[...], 0.0)
        n0_ref[...] = n
        m0_ref[...] = _dot(n, Wes_ref[...])
        ohT = _onehot_t(ngi_ref[...], bn_blk)

        @pl.when(pl.program_id(0) == 0)
        def _():
            sn_ref[...] = jnp.zeros_like(sn_ref)
            ncnt_ref[...] = jnp.zeros_like(ncnt_ref)

        sn_ref[...] += _seg_dot(ohT, n)
        ncnt_ref[...] += _seg_dot(ohT, jnp.ones((bn_blk, 8), F32))

    return pl.pallas_call(
        body,
        grid=(grid,),
        in_specs=[
            pl.BlockSpec((bn_blk, dn), lambda j: (j, 0)),
            pl.BlockSpec((bn_blk, 1), lambda j: (j, 0)),
            pl.BlockSpec((dn, h), lambda j: (0, 0)),
            pl.BlockSpec((1, h), lambda j: (0, 0)),
            pl.BlockSpec((h, he), lambda j: (0, 0)),
        ],
        out_specs=[
            pl.BlockSpec((bn_blk, h), lambda j: (j, 0)),
            pl.BlockSpec((bn_blk, he), lambda j: (j, 0)),
            pl.BlockSpec((GB, h), lambda j: (0, 0)),
            pl.BlockSpec((GB, 8), lambda j: (0, 0)),
        ],
        out_shape=[
            jax.ShapeDtypeStruct((n_total, h), F32),
            jax.ShapeDtypeStruct((n_total, he), F32),
            jax.ShapeDtypeStruct((GB, h), F32),
            jax.ShapeDtypeStruct((GB, 8), F32),
        ],
    )(nf, ngi_col, Wn, bn_row, Wes0)


def _tc_edge(e, gm, Wee, be_row, be_blk):
    e_total, he = e.shape
    grid = e_total // be_blk

    def body(e_ref, gm_ref, Wee_ref, be_ref, out_ref):
        out_ref[...] = jnp.maximum(
            _dot(e_ref[...], Wee_ref[...]) + gm_ref[...] + be_ref[...], 0.0)

    return pl.pallas_call(
        body,
        grid=(grid,),
        in_specs=[
            pl.BlockSpec((be_blk, he), lambda j: (j, 0)),
            pl.BlockSpec((be_blk, he), lambda j: (j, 0)),
            pl.BlockSpec((he, he), lambda j: (0, 0)),
            pl.BlockSpec((1, he), lambda j: (0, 0)),
        ],
        out_specs=pl.BlockSpec((be_blk, he), lambda j: (j, 0)),
        out_shape=jax.ShapeDtypeStruct((e_total, he), F32),
    )(e, gm, Wee, be_row)


def _tc_node(n, p, rc, ngi_col, Wnn, Wni, bn_row, Wes_next, bn_blk):
    n_total, h = n.shape
    he = Wni.shape[0]
    with_m = Wes_next is not None
    grid = n_total // bn_blk

    def body(*refs):
        if with_m:
            (n_ref, p_ref, rc_ref, ngi_ref, Wnn_ref, Wni_ref, bn_ref,
             Wes_ref, n1_ref, sn_ref, ncnt_ref, m1_ref) = refs
        else:
            (n_ref, p_ref, rc_ref, ngi_ref, Wnn_ref, Wni_ref, bn_ref,
             n1_ref, sn_ref, ncnt_ref) = refs
        pv = p_ref[0] + p_ref[1]
        rcv = rc_ref[...]
        cnt = rcv[0, :, :1] + rcv[1, :, :1]
        agg = pv / jnp.maximum(cnt, 1.0)
        x = jnp.maximum(
            _dot(n_ref[...], Wnn_ref[...]) + _dot(agg, Wni_ref[...])
            + bn_ref[...],
            0.0)
        n1_ref[...] = x
        if with_m:
            m1_ref[...] = _dot(x, Wes_ref[...])
        ohT = _onehot_t(ngi_ref[...], bn_blk)

        @pl.when(pl.program_id(0) == 0)
        def _():
            sn_ref[...] = jnp.zeros_like(sn_ref)
            ncnt_ref[...] = jnp.zeros_like(ncnt_ref)

        sn_ref[...] += _seg_dot(ohT, x)
        ncnt_ref[...] += _seg_dot(ohT, jnp.ones((bn_blk, 8), F32))

    in_specs = [
        pl.BlockSpec((bn_blk, h), lambda j: (j, 0)),
        pl.BlockSpec((2, bn_blk, he), lambda j: (0, j, 0)),
        pl.BlockSpec((2, bn_blk, he), lambda j: (0, j, 0)),
        pl.BlockSpec((bn_blk, 1), lambda j: (j, 0)),
        pl.BlockSpec((h, h), lambda j: (0, 0)),
        pl.BlockSpec((he, h), lambda j: (0, 0)),
        pl.BlockSpec((1, h), lambda j: (0, 0)),
    ]
    out_specs = [
        pl.BlockSpec((bn_blk, h), lambda j: (j, 0)),
        pl.BlockSpec((GB, h), lambda j: (0, 0)),
        pl.BlockSpec((GB, 8), lambda j: (0, 0)),
    ]
    out_shape = [
        jax.ShapeDtypeStruct((n_total, h), F32),
        jax.ShapeDtypeStruct((GB, h), F32),
        jax.ShapeDtypeStruct((GB, 8), F32),
    ]
    args = [n, p, rc, ngi_col, Wnn, Wni, bn_row]
    if with_m:
        in_specs.append(pl.BlockSpec((h, he), lambda j: (0, 0)))
        out_specs.append(pl.BlockSpec((bn_blk, he), lambda j: (j, 0)))
        out_shape.append(jax.ShapeDtypeStruct((n_total, he), F32))
        args.append(Wes_next)

    return pl.pallas_call(
        body,
        grid=(grid,),
        in_specs=in_specs,
        out_specs=out_specs,
        out_shape=out_shape,
    )(*args)


def _tc_global(sn, ncnt, q, ecnt, Wgn, Wge, bg_row, g, Wgg, ro):
    hg = Wgn.shape[1]
    has_g = g is not None
    readout = ro is not None

    def body(*refs):
        refs = list(refs)
        sn_ref = refs.pop(0)
        ncnt_ref = refs.pop(0)
        q_ref = refs.pop(0)
        ecnt_ref = refs.pop(0)
        Wgn_ref = refs.pop(0)
        Wge_ref = refs.pop(0)
        bg_ref = refs.pop(0)
        if has_g:
            g_ref = refs.pop(0)
            Wgg_ref = refs.pop(0)
        if readout:
            roW_ref = refs.pop(0)
            rob_ref = refs.pop(0)
        g1_ref = refs.pop(0)
        if readout:
            out_ref = refs.pop(0)

        nc = jnp.maximum(ncnt_ref[...][:, :1], 1.0)
        ec = jnp.maximum(ecnt_ref[...][:, :1], 1.0)
        ng = sn_ref[...] / nc
        eg = (q_ref[0] + q_ref[1]) / ec
        acc = _dot(ng, Wgn_ref[...]) + _dot(eg, Wge_ref[...])
        if has_g:
            acc = acc + _dot(g_ref[...], Wgg_ref[...])
        gn = jnp.maximum(acc + bg_ref[...], 0.0)
        g1_ref[...] = gn
        if readout:
            out_ref[...] = _dot(gn, roW_ref[...]) + rob_ref[...]

    args = [sn, ncnt, q, ecnt, Wgn, Wge, bg_row]
    out_shape = [jax.ShapeDtypeStruct((GB, hg), F32)]
    if has_g:
        args += [g, Wgg]
    if readout:
        args += [ro[0], ro[1]]
        out_shape.append(jax.ShapeDtypeStruct((GB, 128), F32))

    return pl.pallas_call(body, out_shape=out_shape)(*args)


def kernel(node_features, edge_features, senders, receivers,
           node_graph_idx, edge_graph_idx, params):
    p = params
    nf = node_features.astype(F32)
    ef = edge_features.astype(F32)
    snd = senders.astype(I32)
    rcv = receivers.astype(I32)
    ngi = node_graph_idx.astype(I32)
    egi = edge_graph_idx.astype(I32)

    n_total, _ = nf.shape
    e_total, _ = ef.shape
    h = p['enc_Wn'].shape[1]
    he = p['enc_We'].shape[1]
    n_layers = sum(1 for k in p if k.startswith('layer_'))

    be_blk = 4000
    bn_blk = 1000
    chunks = e_total // (NW * IDXW)
    algn = NS * IDXW * 4
    n_acc = ((n_total + algn - 1) // algn) * algn

    snd3d = snd.reshape(NW, chunks, IDXW)
    rcv3d = rcv.reshape(NW, chunks, IDXW)
    egi3d = egi.reshape(NW, chunks, IDXW)
    egi_col = egi.reshape(e_total, 1)
    ngi_col = ngi.reshape(n_total, 1)
    zd = jnp.zeros((IDXW, he), F32)
    onesb = jnp.ones((IDXW, he), F32)
    row = lambda b: b.reshape(1, -1)
    roW = jnp.pad(p['ro_W'], ((0, 0), (0, 127)))
    rob = jnp.pad(p['ro_b'].reshape(1, 1), ((0, 0), (0, 127)))

    e, ecnt = _tc_enc_edge(ef, egi_col, p['enc_We'], row(p['enc_be']), be_blk)
    n, m, sn, ncnt = _tc_enc_node(
        nf, ngi_col, p['enc_Wn'], row(p['enc_bn']),
        p['layer_0']['Wes'], bn_blk)
    q = _sc_scatter(e, egi3d, rcv3d, zd, n_acc, with_recv=False)[0]
    rc = _sc_counts(rcv3d, zd, onesb, n_acc)
    g = _tc_global(sn, ncnt, q, ecnt, p['enc_Wgn'], p['enc_Wge'],
                   row(p['enc_bg']), None, None, None)[0]

    out = None
    for i in range(n_layers):
        lp = p['layer_%d' % i]
        last = i == n_layers - 1
        gm = _sc_gather(m, snd3d)
        e = _tc_edge(e, gm, lp['Wee'], row(lp['be']), be_blk)
        q, pq = _sc_scatter(e, egi3d, rcv3d, zd, n_acc, with_recv=True)
        wes_next = None if last else p['layer_%d' % (i + 1)]['Wes']
        node_out = _tc_node(n, pq, rc, ngi_col, lp['Wnn'], lp['Wni'],
                            row(lp['bn']), wes_next, bn_blk)
        if last:
            n, sn, ncnt = node_out
        else:
            n, sn, ncnt, m = node_out
        glob = _tc_global(sn, ncnt, q, ecnt, lp['Wgn'], lp['Wge'],
                          row(lp['bg']), g, lp['Wgg'],
                          (roW, rob) if last else None)
        if last:
            g, out = glob
        else:
            g = glob[0]

    return out[:, :1]


# speedup vs baseline: 1.0038x; 1.0038x over previous
import jax
import jax.numpy as jnp
from jax import lax
from jax.experimental import pallas as pl
from jax.experimental.pallas import tpu as pltpu
from jax.experimental.pallas import tpu_sc as plsc

F32 = jnp.float32
I32 = jnp.int32
HIGH = lax.Precision.HIGHEST

NC, NS = 2, 16
NW = NC * NS
IDXW = 80
GB = 256


def _sc_mesh():
    return plsc.VectorSubcoreMesh(
        core_axis_name="c", subcore_axis_name="s",
        num_cores=NC, num_subcores=NS)


def _sc_gather(table, idx3d):
    _, chunks, w = idx3d.shape
    e_total = NW * chunks * w
    d = table.shape[1]
    rows_w = chunks * w

    u = 5
    main = chunks // u
    tail = chunks % u

    def body(table_ref, idx_ref, out_ref, idx_v, rows_v, *sems):
        gsems = sems[:u]
        wsem = sems[u]
        wid = lax.axis_index("c") * NS + lax.axis_index("s")
        pltpu.sync_copy(idx_ref.at[wid], idx_v)
        ebase = wid * rows_w

        def group(j2, carry):
            j0 = j2 * u
            gds = [pltpu.async_copy(table_ref.at[idx_v.at[j0 + b]],
                                    rows_v.at[b], gsems[b])
                   for b in range(u)]
            wds = []
            for b in range(u):
                gds[b].wait()
                wds.append(pltpu.async_copy(
                    rows_v.at[b], out_ref.at[pl.ds(ebase + (j0 + b) * w, w)],
                    wsem))
            for d_ in wds:
                d_.wait()
            return carry

        lax.fori_loop(0, main, group, 0)
        for t in range(tail):
            j = main * u + t
            pltpu.async_copy(table_ref.at[idx_v.at[j]], rows_v.at[0],
                             gsems[0]).wait()
            pltpu.sync_copy(rows_v.at[0], out_ref.at[pl.ds(ebase + j * w, w)])

    return pl.kernel(
        body,
        out_type=jax.ShapeDtypeStruct((e_total, d), F32),
        mesh=_sc_mesh(),
        scratch_types=[
            pltpu.VMEM((chunks, w), I32),
            pltpu.VMEM((u, w, d), F32),
        ] + [pltpu.SemaphoreType.DMA] * (u + 1),
    )(table, idx3d)


def _sc_scatter(e, egi3d, rcv3d, zd, n_acc, with_recv):
    _, chunks, w = egi3d.shape
    d = e.shape[1]
    rows_w = chunks * w
    npt = n_acc // NS
    nz = npt // w
    gpt = GB // NS

    u = 2 if with_recv else 5
    main = chunks // u
    tail = chunks % u

    out_type = [jax.ShapeDtypeStruct((NC, GB, d), F32)]
    scratch = [
        pltpu.VMEM((u, w, d), F32),
        pltpu.VMEM((u, 1, w), I32),
        pltpu.VMEM_SHARED((GB, d), F32),
    ]
    if with_recv:
        out_type.append(jax.ShapeDtypeStruct((NC, n_acc, d), F32))
        scratch += [pltpu.VMEM((u, 1, w), I32),
                    pltpu.VMEM_SHARED((n_acc, d), F32)]
    scratch += [pltpu.SemaphoreType.DMA] * (2 * u)

    def body(e_ref, egi_ref, rcv_ref, zd_ref, *rest):
        rest = list(rest)
        q_ref = rest.pop(0)
        p_ref = rest.pop(0) if with_recv else None
        rows = rest.pop(0)
        idx_g = rest.pop(0)
        acc_g = rest.pop(0)
        if with_recv:
            idx_r = rest.pop(0)
            acc_n = rest.pop(0)
        lsems = rest[:u]
        asems = rest[u:2 * u]

        cid = lax.axis_index("c")
        sid = lax.axis_index("s")
        wid = cid * NS + sid

        pltpu.sync_copy(zd_ref, rows.at[0])
        pltpu.sync_copy(rows.at[0, pl.ds(0, gpt)],
                        acc_g.at[pl.ds(sid * gpt, gpt)])
        if with_recv:
            for z in range(nz):
                pltpu.sync_copy(rows.at[0], acc_n.at[pl.ds(sid * npt + z * w, w)])
        plsc.subcore_barrier()

        ebase = wid * rows_w

        def issue_loads(j, b):
            lds = [pltpu.async_copy(e_ref.at[pl.ds(ebase + j * w, w)],
                                    rows.at[b], lsems[b]),
                   pltpu.async_copy(egi_ref.at[wid, pl.ds(j, 1)],
                                    idx_g.at[b], lsems[b])]
            if with_recv:
                lds.append(pltpu.async_copy(rcv_ref.at[wid, pl.ds(j, 1)],
                                            idx_r.at[b], lsems[b]))
            return lds

        def issue_adds(b):
            ads = [pltpu.async_copy(rows.at[b], acc_g.at[idx_g.at[b, 0]],
                                    asems[b], add=True)]
            if with_recv:
                ads.append(pltpu.async_copy(rows.at[b],
                                            acc_n.at[idx_r.at[b, 0]],
                                            asems[b], add=True))
            return ads

        def group(j2, carry):
            j0 = j2 * u
            lds = [issue_loads(j0 + b, b) for b in range(u)]
            ads = []
            for b in range(u):
                for l_ in lds[b]:
                    l_.wait()
                ads += issue_adds(b)
            for a_ in ads:
                a_.wait()
            return carry

        lax.fori_loop(0, main, group, 0)
        for t in range(tail):
            for l_ in issue_loads(main * u + t, 0):
                l_.wait()
            for a_ in issue_adds(0):
                a_.wait()
        plsc.subcore_barrier()

        pltpu.sync_copy(acc_g.at[pl.ds(sid * gpt, gpt)],
                        rows.at[0, pl.ds(0, gpt)])
        pltpu.sync_copy(rows.at[0, pl.ds(0, gpt)],
                        q_ref.at[cid, pl.ds(sid * gpt, gpt)])
        if with_recv:
            for z in range(nz):
                r0 = sid * npt + z * w
                pltpu.sync_copy(acc_n.at[pl.ds(r0, w)], rows.at[0])
                pltpu.sync_copy(rows.at[0], p_ref.at[cid, pl.ds(r0, w)])

    res = pl.kernel(
        body,
        out_type=out_type,
        mesh=_sc_mesh(),
        scratch_types=scratch,
    )(e, egi3d, rcv3d, zd)
    return res


def _sc_counts(rcv3d, zd, onesb, n_acc):
    _, chunks, w = rcv3d.shape
    d = onesb.shape[1]
    npt = n_acc // NS
    nz = npt // w

    def body(rcv_ref, zd_ref, ones_ref, rc_ref, idx_r, buf, acc_c):
        cid = lax.axis_index("c")
        sid = lax.axis_index("s")
        wid = cid * NS + sid
        pltpu.sync_copy(zd_ref, buf)
        for z in range(nz):
            pltpu.sync_copy(buf, acc_c.at[pl.ds(sid * npt + z * w, w)])
        pltpu.sync_copy(ones_ref, buf)
        pltpu.sync_copy(rcv_ref.at[wid], idx_r)
        plsc.subcore_barrier()

        def step(j, carry):
            pltpu.sync_copy(buf, acc_c.at[idx_r.at[j]], add=True)
            return carry

        lax.fori_loop(0, chunks, step, 0)
        plsc.subcore_barrier()
        for z in range(nz):
            r0 = sid * npt + z * w
            pltpu.sync_copy(acc_c.at[pl.ds(r0, w)], buf)
            pltpu.sync_copy(buf, rc_ref.at[cid, pl.ds(r0, w)])

    return pl.kernel(
        body,
        out_type=jax.ShapeDtypeStruct((NC, n_acc, d), F32),
        mesh=_sc_mesh(),
        scratch_types=[
            pltpu.VMEM((chunks, w), I32),
            pltpu.VMEM((w, d), F32),
            pltpu.VMEM_SHARED((n_acc, d), F32),
        ],
    )(rcv3d, zd, onesb)


def _dot(a, b):
    return jnp.dot(a, b, preferred_element_type=F32)


def _onehot_t(ids, nrows):
    return (jnp.broadcast_to(ids, (nrows, GB))
            == lax.broadcasted_iota(I32, (nrows, GB), 1)).astype(F32)


def _seg_dot(ohT, x):
    return lax.dot_general(ohT, x, (((0,), (0,)), ((), ())),
                           preferred_element_type=F32, precision=HIGH)


def _tc_enc_edge(ef, egi_col, We, be_row, be_blk):
    e_total, de = ef.shape
    he = We.shape[1]
    grid = e_total // be_blk

    def body(ef_ref, egi_ref, We_ref, be_ref, e0_ref, ecnt_ref):
        x = _dot(ef_ref[...], We_ref[...])
        e0_ref[...] = jnp.maximum(x + be_ref[...], 0.0)
        ohT = _onehot_t(egi_ref[...], be_blk)
        cnt = _seg_dot(ohT, jnp.ones((be_blk, 8), F32))

        @pl.when(pl.program_id(0) == 0)
        def _():
            ecnt_ref[...] = jnp.zeros_like(ecnt_ref)

        ecnt_ref[...] += cnt

    return pl.pallas_call(
        body,
        grid=(grid,),
        in_specs=[
            pl.BlockSpec((be_blk, de), lambda j: (j, 0)),
            pl.BlockSpec((be_blk, 1), lambda j: (j, 0)),
            pl.BlockSpec((de, he), lambda j: (0, 0)),
            pl.BlockSpec((1, he), lambda j: (0, 0)),
        ],
        out_specs=[
            pl.BlockSpec((be_blk, he), lambda j: (j, 0)),
            pl.BlockSpec((GB, 8), lambda j: (0, 0)),
        ],
        out_shape=[
            jax.ShapeDtypeStruct((e_total, he), F32),
            jax.ShapeDtypeStruct((GB, 8), F32),
        ],
    )(ef, egi_col, We, be_row)


def _tc_enc_node(nf, ngi_col, Wn, bn_row, Wes0, bn_blk):
    n_total, dn = nf.shape
    h = Wn.shape[1]
    he = Wes0.shape[1]
    grid = n_total // bn_blk

    def body(nf_ref, ngi_ref, Wn_ref, bn_ref, Wes_ref,
             n0_ref, m0_ref, sn_ref, ncnt_ref):
        n = jnp.maximum(_dot(nf_ref[...], Wn_ref[...]) + bn_ref[...], 0.0)
        n0_ref[...] = n
        m0_ref[...] = _dot(n, Wes_ref[...])
        ohT = _onehot_t(ngi_ref[...], bn_blk)

        @pl.when(pl.program_id(0) == 0)
        def _():
            sn_ref[...] = jnp.zeros_like(sn_ref)
            ncnt_ref[...] = jnp.zeros_like(ncnt_ref)

        sn_ref[...] += _seg_dot(ohT, n)
        ncnt_ref[...] += _seg_dot(ohT, jnp.ones((bn_blk, 8), F32))

    return pl.pallas_call(
        body,
        grid=(grid,),
        in_specs=[
            pl.BlockSpec((bn_blk, dn), lambda j: (j, 0)),
            pl.BlockSpec((bn_blk, 1), lambda j: (j, 0)),
            pl.BlockSpec((dn, h), lambda j: (0, 0)),
            pl.BlockSpec((1, h), lambda j: (0, 0)),
            pl.BlockSpec((h, he), lambda j: (0, 0)),
        ],
        out_specs=[
            pl.BlockSpec((bn_blk, h), lambda j: (j, 0)),
            pl.BlockSpec((bn_blk, he), lambda j: (j, 0)),
            pl.BlockSpec((GB, h), lambda j: (0, 0)),
            pl.BlockSpec((GB, 8), lambda j: (0, 0)),
        ],
        out_shape=[
            jax.ShapeDtypeStruct((n_total, h), F32),
            jax.ShapeDtypeStruct((n_total, he), F32),
            jax.ShapeDtypeStruct((GB, h), F32),
            jax.ShapeDtypeStruct((GB, 8), F32),
        ],
    )(nf, ngi_col, Wn, bn_row, Wes0)


def _tc_edge(e, gm, Wee, be_row, be_blk):
    e_total, he = e.shape
    grid = e_total // be_blk

    def body(e_ref, gm_ref, Wee_ref, be_ref, out_ref):
        out_ref[...] = jnp.maximum(
            _dot(e_ref[...], Wee_ref[...]) + gm_ref[...] + be_ref[...], 0.0)

    return pl.pallas_call(
        body,
        grid=(grid,),
        in_specs=[
            pl.BlockSpec((be_blk, he), lambda j: (j, 0)),
            pl.BlockSpec((be_blk, he), lambda j: (j, 0)),
            pl.BlockSpec((he, he), lambda j: (0, 0)),
            pl.BlockSpec((1, he), lambda j: (0, 0)),
        ],
        out_specs=pl.BlockSpec((be_blk, he), lambda j: (j, 0)),
        out_shape=jax.ShapeDtypeStruct((e_total, he), F32),
    )(e, gm, Wee, be_row)


def _tc_node(n, p, rc, ngi_col, Wnn, Wni, bn_row, Wes_next, bn_blk):
    n_total, h = n.shape
    he = Wni.shape[0]
    with_m = Wes_next is not None
    grid = n_total // bn_blk

    def body(*refs):
        if with_m:
            (n_ref, p_ref, rc_ref, ngi_ref, Wnn_ref, Wni_ref, bn_ref,
             Wes_ref, n1_ref, sn_ref, ncnt_ref, m1_ref) = refs
        else:
            (n_ref, p_ref, rc_ref, ngi_ref, Wnn_ref, Wni_ref, bn_ref,
             n1_ref, sn_ref, ncnt_ref) = refs
        pv = p_ref[0] + p_ref[1]
        rcv = rc_ref[...]
        cnt = rcv[0, :, :1] + rcv[1, :, :1]
        agg = pv / jnp.maximum(cnt, 1.0)
        x = jnp.maximum(
            _dot(n_ref[...], Wnn_ref[...]) + _dot(agg, Wni_ref[...])
            + bn_ref[...],
            0.0)
        n1_ref[...] = x
        if with_m:
            m1_ref[...] = _dot(x, Wes_ref[...])
        ohT = _onehot_t(ngi_ref[...], bn_blk)

        @pl.when(pl.program_id(0) == 0)
        def _():
            sn_ref[...] = jnp.zeros_like(sn_ref)
            ncnt_ref[...] = jnp.zeros_like(ncnt_ref)

        sn_ref[...] += _seg_dot(ohT, x)
        ncnt_ref[...] += _seg_dot(ohT, jnp.ones((bn_blk, 8), F32))

    in_specs = [
        pl.BlockSpec((bn_blk, h), lambda j: (j, 0)),
        pl.BlockSpec((2, bn_blk, he), lambda j: (0, j, 0)),
        pl.BlockSpec((2, bn_blk, he), lambda j: (0, j, 0)),
        pl.BlockSpec((bn_blk, 1), lambda j: (j, 0)),
        pl.BlockSpec((h, h), lambda j: (0, 0)),
        pl.BlockSpec((he, h), lambda j: (0, 0)),
        pl.BlockSpec((1, h), lambda j: (0, 0)),
    ]
    out_specs = [
        pl.BlockSpec((bn_blk, h), lambda j: (j, 0)),
        pl.BlockSpec((GB, h), lambda j: (0, 0)),
        pl.BlockSpec((GB, 8), lambda j: (0, 0)),
    ]
    out_shape = [
        jax.ShapeDtypeStruct((n_total, h), F32),
        jax.ShapeDtypeStruct((GB, h), F32),
        jax.ShapeDtypeStruct((GB, 8), F32),
    ]
    args = [n, p, rc, ngi_col, Wnn, Wni, bn_row]
    if with_m:
        in_specs.append(pl.BlockSpec((h, he), lambda j: (0, 0)))
        out_specs.append(pl.BlockSpec((bn_blk, he), lambda j: (j, 0)))
        out_shape.append(jax.ShapeDtypeStruct((n_total, he), F32))
        args.append(Wes_next)

    return pl.pallas_call(
        body,
        grid=(grid,),
        in_specs=in_specs,
        out_specs=out_specs,
        out_shape=out_shape,
    )(*args)


def _tc_global(sn, ncnt, q, ecnt, Wgn, Wge, bg_row, g, Wgg, ro):
    hg = Wgn.shape[1]
    has_g = g is not None
    readout = ro is not None

    def body(*refs):
        refs = list(refs)
        sn_ref = refs.pop(0)
        ncnt_ref = refs.pop(0)
        q_ref = refs.pop(0)
        ecnt_ref = refs.pop(0)
        Wgn_ref = refs.pop(0)
        Wge_ref = refs.pop(0)
        bg_ref = refs.pop(0)
        if has_g:
            g_ref = refs.pop(0)
            Wgg_ref = refs.pop(0)
        if readout:
            roW_ref = refs.pop(0)
            rob_ref = refs.pop(0)
        g1_ref = refs.pop(0)
        if readout:
            out_ref = refs.pop(0)

        nc = jnp.maximum(ncnt_ref[...][:, :1], 1.0)
        ec = jnp.maximum(ecnt_ref[...][:, :1], 1.0)
        ng = sn_ref[...] / nc
        eg = (q_ref[0] + q_ref[1]) / ec
        acc = _dot(ng, Wgn_ref[...]) + _dot(eg, Wge_ref[...])
        if has_g:
            acc = acc + _dot(g_ref[...], Wgg_ref[...])
        gn = jnp.maximum(acc + bg_ref[...], 0.0)
        g1_ref[...] = gn
        if readout:
            out_ref[...] = _dot(gn, roW_ref[...]) + rob_ref[...]

    args = [sn, ncnt, q, ecnt, Wgn, Wge, bg_row]
    out_shape = [jax.ShapeDtypeStruct((GB, hg), F32)]
    if has_g:
        args += [g, Wgg]
    if readout:
        args += [ro[0], ro[1]]
        out_shape.append(jax.ShapeDtypeStruct((GB, 128), F32))

    return pl.pallas_call(body, out_shape=out_shape)(*args)


def kernel(node_features, edge_features, senders, receivers,
           node_graph_idx, edge_graph_idx, params):
    p = params
    nf = node_features.astype(F32)
    ef = edge_features.astype(F32)
    snd = senders.astype(I32)
    rcv = receivers.astype(I32)
    ngi = node_graph_idx.astype(I32)
    egi = edge_graph_idx.astype(I32)

    n_total, _ = nf.shape
    e_total, _ = ef.shape
    h = p['enc_Wn'].shape[1]
    he = p['enc_We'].shape[1]
    n_layers = sum(1 for k in p if k.startswith('layer_'))

    be_blk = 4000
    bn_blk = 1000
    chunks = e_total // (NW * IDXW)
    algn = NS * IDXW * 4
    n_acc = ((n_total + algn - 1) // algn) * algn

    snd3d = snd.reshape(NW, chunks, IDXW)
    rcv3d = rcv.reshape(NW, chunks, IDXW)
    egi3d = egi.reshape(NW, chunks, IDXW)
    egi_col = egi.reshape(e_total, 1)
    ngi_col = ngi.reshape(n_total, 1)
    zd = jnp.zeros((IDXW, he), F32)
    onesb = jnp.ones((IDXW, he), F32)
    row = lambda b: b.reshape(1, -1)
    roW = jnp.pad(p['ro_W'], ((0, 0), (0, 127)))
    rob = jnp.pad(p['ro_b'].reshape(1, 1), ((0, 0), (0, 127)))

    e, ecnt = _tc_enc_edge(ef, egi_col, p['enc_We'], row(p['enc_be']), be_blk)
    n, m, sn, ncnt = _tc_enc_node(
        nf, ngi_col, p['enc_Wn'], row(p['enc_bn']),
        p['layer_0']['Wes'], bn_blk)
    q = _sc_scatter(e, egi3d, rcv3d, zd, n_acc, with_recv=False)[0]
    rc = _sc_counts(rcv3d, zd, onesb, n_acc)
    g = _tc_global(sn, ncnt, q, ecnt, p['enc_Wgn'], p['enc_Wge'],
                   row(p['enc_bg']), None, None, None)[0]

    out = None
    for i in range(n_layers):
        lp = p['layer_%d' % i]
        last = i == n_layers - 1
        gm = _sc_gather(m, snd3d)
        e = _tc_edge(e, gm, lp['Wee'], row(lp['be']), be_blk)
        q, pq = _sc_scatter(e, egi3d, rcv3d, zd, n_acc, with_recv=True)
        wes_next = None if last else p['layer_%d' % (i + 1)]['Wes']
        node_out = _tc_node(n, pq, rc, ngi_col, lp['Wnn'], lp['Wni'],
                            row(lp['bn']), wes_next, bn_blk)
        if last:
            n, sn, ncnt = node_out
        else:
            n, sn, ncnt, m = node_out
        glob = _tc_global(sn, ncnt, q, ecnt, lp['Wgn'], lp['Wge'],
                          row(lp['bg']), g, lp['Wgg'],
                          (roW, rob) if last else None)
        if last:
            g, out = glob
        else:
            g = glob[0]

    return out[:, :1]
